# Initial kernel scaffold; baseline (speedup 1.0000x reference)
import jax
import jax.numpy as jnp
from jax.experimental import pallas as pl


def kernel(features, w1, b1, w2, b2, w3, b3):
    raise NotImplementedError("write your pallas kernel here")


# speedup vs baseline: 1.3945x; 1.3945x over previous
import jax
import jax.numpy as jnp
from jax import lax
from jax.experimental import pallas as pl
from jax.experimental.pallas import tpu as pltpu

_FEAT = 30
_PACK = 128
_PACKED = _FEAT * _PACK
_TB = 256


def _round_up(x, m):
    return ((x + m - 1) // m) * m


def _affine_tanh_kernel(x_ref, w_ref, c_ref, o_ref):
    y = lax.dot_general(
        x_ref[...], w_ref[...],
        dimension_numbers=(((1,), (0,)), ((), ())),
        preferred_element_type=jnp.float32,
    )
    o_ref[...] = jnp.tanh(y + c_ref[0])


def kernel(features, w1, b1, w2, b2, w3, b3):
    B = features.shape[0]
    x = features.astype(jnp.float32)

    w_row = (w3 @ w2 @ w1).reshape(_FEAT).astype(jnp.float32)
    c = (b1 @ w2.T @ w3.T + b2 @ w3.T + b3).reshape(1).astype(jnp.float32)

    B_pad = _round_up(B, _PACK * 8)
    if B_pad != B:
        x = jnp.pad(x, ((0, B_pad - B), (0, 0)))
    rows = B_pad // _PACK
    x_pack = x.reshape(rows, _PACKED)

    w_pack = (jnp.eye(_PACK, dtype=jnp.float32)[:, None, :]
              * w_row[None, :, None]).reshape(_PACKED, _PACK)

    if rows > _TB:
        tb = _TB
    elif rows >= 16:
        tb = _round_up(pl.cdiv(rows, 2), 8)
    else:
        tb = rows
    num_tiles = pl.cdiv(rows, tb)

    out = pl.pallas_call(
        _affine_tanh_kernel,
        out_shape=jax.ShapeDtypeStruct((rows, _PACK), jnp.float32),
        grid=(num_tiles,),
        in_specs=[
            pl.BlockSpec((tb, _PACKED), lambda i: (i, 0)),
            pl.BlockSpec((_PACKED, _PACK), lambda i: (0, 0)),
            pl.BlockSpec(memory_space=pltpu.MemorySpace.SMEM),
        ],
        out_specs=pl.BlockSpec((tb, _PACK), lambda i: (i, 0)),
        compiler_params=pltpu.CompilerParams(
            dimension_semantics=("parallel",),
        ),
    )(x_pack, w_pack, c)

    return out.reshape(B_pad, 1)[:B]


# speedup vs baseline: 1.3987x; 1.0030x over previous
import jax
import jax.numpy as jnp
from jax import lax
from jax.experimental import pallas as pl
from jax.experimental.pallas import tpu as pltpu

_FEAT = 30
_PACK = 128
_PACKED = _FEAT * _PACK
_TB = 256


def _round_up(x, m):
    return ((x + m - 1) // m) * m


def _affine_tanh_kernel(x_ref, w_ref, c_ref, o_ref):
    y = lax.dot_general(
        x_ref[...], w_ref[...],
        dimension_numbers=(((1,), (0,)), ((), ())),
        preferred_element_type=jnp.float32,
    )
    tb = y.shape[0]
    o_ref[...] = jnp.tanh(y + c_ref[0]).reshape(tb // 8, 8, _PACK)


def kernel(features, w1, b1, w2, b2, w3, b3):
    B = features.shape[0]
    x = features.astype(jnp.float32)

    w_row = (w3 @ w2 @ w1).reshape(_FEAT).astype(jnp.float32)
    c = (b1 @ w2.T @ w3.T + b2 @ w3.T + b3).reshape(1).astype(jnp.float32)

    B_pad = _round_up(B, _PACK * 8)
    if B_pad != B:
        x = jnp.pad(x, ((0, B_pad - B), (0, 0)))
    rows = B_pad // _PACK
    x_pack = x.reshape(rows, _PACKED)

    w_pack = (jnp.eye(_PACK, dtype=jnp.float32)[:, None, :]
              * w_row[None, :, None]).reshape(_PACKED, _PACK)

    if rows > _TB:
        tb = _TB
    elif rows >= 16:
        tb = _round_up(pl.cdiv(rows, 2), 8)
    else:
        tb = rows
    num_tiles = pl.cdiv(rows, tb)

    out = pl.pallas_call(
        _affine_tanh_kernel,
        out_shape=jax.ShapeDtypeStruct((rows // 8, 8, _PACK), jnp.float32),
        grid=(num_tiles,),
        in_specs=[
            pl.BlockSpec((tb, _PACKED), lambda i: (i, 0)),
            pl.BlockSpec((_PACKED, _PACK), lambda i: (0, 0)),
            pl.BlockSpec(memory_space=pltpu.MemorySpace.SMEM),
        ],
        out_specs=pl.BlockSpec((tb // 8, 8, _PACK), lambda i: (i, 0, 0)),
        compiler_params=pltpu.CompilerParams(
            dimension_semantics=("parallel",),
        ),
    )(x_pack, w_pack, c)

    return out.reshape(B_pad, 1)[:B]
